# Initial kernel scaffold; baseline (speedup 1.0000x reference)
import jax
import jax.numpy as jnp
from jax.experimental import pallas as pl


def kernel(duration_bins, table):
    raise NotImplementedError("write your pallas kernel here")


# speedup vs baseline: 3.3968x; 3.3968x over previous
import functools

import jax
import jax.numpy as jnp
from jax import lax
from jax.experimental import pallas as pl
from jax.experimental.pallas import tpu as pltpu
from jax.experimental.pallas import tpu_sc as plsc

NUM_BINS = 64
EMB_DIM = 64
BATCH = 4096
SEQ = 200

NC = 2
NS = 16
NW = NC * NS

B = BATCH * SEQ
P = B // 2
P_PER_W = P // NW
CHUNK = 128
NSTEP = P_PER_W // CHUNK


def _body(idx_hbm, table2_hbm, out_hbm, idx_v, rows_v, gsem, ssem):
    wid = lax.axis_index("s") * NC + lax.axis_index("c")
    base = wid * P_PER_W

    pltpu.sync_copy(idx_hbm.at[wid], idx_v)

    def step(j, carry):
        pltpu.async_copy(table2_hbm.at[idx_v.at[j]], rows_v, gsem).wait()
        pltpu.async_copy(rows_v, out_hbm.at[pl.ds(base + j * CHUNK, CHUNK)],
                         ssem).wait()
        return carry

    lax.fori_loop(0, NSTEP, step, 0)


@functools.partial(jax.jit, static_argnames=())
def kernel(duration_bins, table):
    pair_table = jnp.concatenate(
        [jnp.repeat(table, NUM_BINS, axis=0), jnp.tile(table, (NUM_BINS, 1))],
        axis=1)
    idx = duration_bins.reshape(-1).astype(jnp.int32)
    pair_idx = (idx[0::2] * NUM_BINS + idx[1::2]).reshape(NW, NSTEP, CHUNK)
    mesh = plsc.VectorSubcoreMesh(core_axis_name="c", subcore_axis_name="s")
    out = pl.kernel(
        _body,
        out_type=jax.ShapeDtypeStruct((P, 2 * EMB_DIM), jnp.float32),
        mesh=mesh,
        scratch_types=[
            pltpu.VMEM((NSTEP, CHUNK), jnp.int32),
            pltpu.VMEM((CHUNK, 2 * EMB_DIM), jnp.float32),
            pltpu.SemaphoreType.DMA,
            pltpu.SemaphoreType.DMA,
        ],
    )(pair_idx, pair_table)
    return out.reshape(BATCH, SEQ, EMB_DIM)


# speedup vs baseline: 4.0819x; 1.2017x over previous
import functools

import jax
import jax.numpy as jnp
from jax import lax
from jax.experimental import pallas as pl
from jax.experimental.pallas import tpu as pltpu
from jax.experimental.pallas import tpu_sc as plsc

NUM_BINS = 64
EMB_DIM = 64
BATCH = 4096
SEQ = 200

NC = 2
NS = 16
NW = NC * NS

B = BATCH * SEQ
P = B // 2
P_PER_W = P // NW
CHUNK = 128
NSTEP = P_PER_W // CHUNK
NBUF = 4


def _body(idx_hbm, table2_hbm, out_hbm, shared, idx_v, bufs, stage_sem,
          gsem, ssem):
    sid = lax.axis_index("s")
    wid = sid * NC + lax.axis_index("c")
    base = wid * P_PER_W

    @pl.when(sid == 0)
    def _():
        pltpu.async_copy(table2_hbm, shared, stage_sem).wait()

    pltpu.sync_copy(idx_hbm.at[wid], idx_v)
    plsc.subcore_barrier()

    for b in range(NBUF):
        pltpu.async_copy(shared.at[idx_v.at[b]], bufs[b], gsem)

    def step(jj, carry):
        for b in range(NBUF):
            j = jj * NBUF + b
            pltpu.make_async_copy(shared.at[idx_v.at[j]], bufs[b], gsem).wait()
            pltpu.async_copy(
                bufs[b], out_hbm.at[pl.ds(base + j * CHUNK, CHUNK)], ssem)
            @pl.when(jj < NSTEP // NBUF - 1)
            def _():
                pltpu.make_async_copy(
                    bufs[b], out_hbm.at[pl.ds(base + j * CHUNK, CHUNK)],
                    ssem).wait()
                pltpu.async_copy(shared.at[idx_v.at[j + NBUF]], bufs[b], gsem)
        return carry

    lax.fori_loop(0, NSTEP // NBUF, step, 0)

    for b in range(NBUF):
        pltpu.make_async_copy(
            bufs[b], out_hbm.at[pl.ds(base, CHUNK)], ssem).wait()


@functools.partial(jax.jit, static_argnames=())
def kernel(duration_bins, table):
    pair_table = jnp.concatenate(
        [jnp.repeat(table, NUM_BINS, axis=0), jnp.tile(table, (NUM_BINS, 1))],
        axis=1)
    idx = duration_bins.reshape(-1).astype(jnp.int32)
    pair_idx = (idx[0::2] * NUM_BINS + idx[1::2]).reshape(NW, NSTEP, CHUNK)
    mesh = plsc.VectorSubcoreMesh(core_axis_name="c", subcore_axis_name="s")
    out = pl.kernel(
        _body,
        out_type=jax.ShapeDtypeStruct((P, 2 * EMB_DIM), jnp.float32),
        mesh=mesh,
        scratch_types=[
            pltpu.VMEM_SHARED((NUM_BINS * NUM_BINS, 2 * EMB_DIM),
                              jnp.float32),
            pltpu.VMEM((NSTEP, CHUNK), jnp.int32),
            [pltpu.VMEM((CHUNK, 2 * EMB_DIM), jnp.float32)
             for _ in range(NBUF)],
            pltpu.SemaphoreType.DMA,
            pltpu.SemaphoreType.DMA,
            pltpu.SemaphoreType.DMA,
        ],
    )(pair_idx, pair_table)
    return out.reshape(BATCH, SEQ, EMB_DIM)


# speedup vs baseline: 4.9153x; 1.2042x over previous
import functools

import jax
import jax.numpy as jnp
from jax import lax
from jax.experimental import pallas as pl
from jax.experimental.pallas import tpu as pltpu
from jax.experimental.pallas import tpu_sc as plsc

NUM_BINS = 64
EMB_DIM = 64
BATCH = 4096
SEQ = 200

NC = 2
NS = 16
NW = NC * NS
L = 16

B = BATCH * SEQ
P = B // 2
P_PER_W = P // NW
CHUNK = 128
NSTEP = P_PER_W // CHUNK
NBUF = 4
ROWS_PER_SUB = NUM_BINS * NUM_BINS // NS


def _body(idx_hbm, table_hbm, out_hbm, shared, raw_v, pidx_v, tab_v, bufs,
          stage_sem, gsem, ssem):
    sid = lax.axis_index("s")
    wid = sid * NC + lax.axis_index("c")
    base = wid * P_PER_W

    pltpu.sync_copy(table_hbm, tab_v)

    for half in range(2):
        row0 = sid * ROWS_PER_SUB + half * (ROWS_PER_SUB // 2)

        def build(r, c, _row0=row0, _half=half):
            row = _row0 + r
            i = lax.shift_right_logical(row, 6)
            j = lax.bitwise_and(row, 63)
            for c4 in range(4):
                bufs[_half][r, pl.ds(c4 * L, L)] = tab_v[i, pl.ds(c4 * L, L)]
                bufs[_half][r, pl.ds(EMB_DIM + c4 * L, L)] = (
                    tab_v[j, pl.ds(c4 * L, L)])
            return c

        lax.fori_loop(0, ROWS_PER_SUB // 2, build, 0)
        pltpu.async_copy(
            bufs[half],
            shared.at[pl.ds(row0, ROWS_PER_SUB // 2)], stage_sem)

    lanes = lax.iota(jnp.int32, L)
    wvec = jnp.where(lax.bitwise_and(lanes, 1) == 0, NUM_BINS, 1)
    swap = lax.bitwise_xor(lanes, 1)
    compress = lax.shift_left(lax.bitwise_and(lanes, 7), 1)
    def _pairs8(vec):
        c = vec * wvec
        s = c + c.at[swap].get(mode="promise_in_bounds")
        return s.at[compress].get(mode="promise_in_bounds")

    for q4 in range(4):
        pltpu.sync_copy(idx_hbm.at[wid, q4], raw_v)

        def pair(j, c, _j0=q4 * (NSTEP // 4)):
            for b8 in range(CHUNK // L):
                row, col = j * 2 + b8 // 4, (b8 % 4) * 2 * L
                ga = _pairs8(raw_v[row, pl.ds(col, L)])
                gb = _pairs8(raw_v[row, pl.ds(col + L, L)])
                pidx_v[_j0 + j, pl.ds(b8 * L, L)] = jnp.where(
                    lanes < 8, ga, gb)
            return c

        lax.fori_loop(0, NSTEP // 4, pair, 0)

    for half in range(2):
        pltpu.make_async_copy(
            bufs[half],
            shared.at[pl.ds(sid * ROWS_PER_SUB, ROWS_PER_SUB // 2)],
            stage_sem).wait()
    plsc.subcore_barrier()

    for b in range(NBUF):
        pltpu.async_copy(shared.at[pidx_v.at[b]], bufs[b], gsem)

    def step(jj, carry):
        for b in range(NBUF):
            j = jj * NBUF + b
            pltpu.make_async_copy(shared.at[pidx_v.at[j]], bufs[b],
                                  gsem).wait()
            pltpu.async_copy(
                bufs[b], out_hbm.at[pl.ds(base + j * CHUNK, CHUNK)], ssem)

            @pl.when(jj < NSTEP // NBUF - 1)
            def _():
                pltpu.make_async_copy(
                    bufs[b], out_hbm.at[pl.ds(base + j * CHUNK, CHUNK)],
                    ssem).wait()
                pltpu.async_copy(shared.at[pidx_v.at[j + NBUF]], bufs[b],
                                 gsem)
        return carry

    lax.fori_loop(0, NSTEP // NBUF, step, 0)

    for b in range(NBUF):
        pltpu.make_async_copy(
            bufs[b], out_hbm.at[pl.ds(base, CHUNK)], ssem).wait()


@functools.partial(jax.jit, static_argnames=())
def kernel(duration_bins, table):
    idx = duration_bins.astype(jnp.int32).reshape(
        NW, 4, P_PER_W // CHUNK // 2, CHUNK)
    mesh = plsc.VectorSubcoreMesh(core_axis_name="c", subcore_axis_name="s")
    out = pl.kernel(
        _body,
        out_type=jax.ShapeDtypeStruct((P, 2 * EMB_DIM), jnp.float32),
        mesh=mesh,
        scratch_types=[
            pltpu.VMEM_SHARED((NUM_BINS * NUM_BINS, 2 * EMB_DIM),
                              jnp.float32),
            pltpu.VMEM((P_PER_W // CHUNK // 2, CHUNK), jnp.int32),
            pltpu.VMEM((NSTEP, CHUNK), jnp.int32),
            pltpu.VMEM((NUM_BINS, EMB_DIM), jnp.float32),
            [pltpu.VMEM((CHUNK, 2 * EMB_DIM), jnp.float32)
             for _ in range(NBUF)],
            pltpu.SemaphoreType.DMA,
            pltpu.SemaphoreType.DMA,
            pltpu.SemaphoreType.DMA,
        ],
    )(idx, table)
    return out.reshape(BATCH, SEQ, EMB_DIM)


# speedup vs baseline: 4.9545x; 1.0080x over previous
import functools

import jax
import jax.numpy as jnp
from jax import lax
from jax.experimental import pallas as pl
from jax.experimental.pallas import tpu as pltpu
from jax.experimental.pallas import tpu_sc as plsc

NUM_BINS = 64
EMB_DIM = 64
BATCH = 4096
SEQ = 200

NC = 2
NS = 16
NW = NC * NS
L = 16

ROWS_W = BATCH // NW
PAIR_R = SEQ // 2
PIDX_STRIDE = 112
P = BATCH * PAIR_R
P_PER_W = ROWS_W * PAIR_R
NSTEP = ROWS_W // 2
TBL_ROWS_SUB = NUM_BINS * NUM_BINS // NS


def _body(idx_hbm, table_hbm, out_hbm, shared, raw_v, pidx_v, tab_v, bufs,
          stage_sem, gsem, ssem):
    sid = lax.axis_index("s")
    wid = sid * NC + lax.axis_index("c")
    base = wid * P_PER_W

    pltpu.sync_copy(table_hbm, tab_v)

    for half in range(2):
        row0 = sid * TBL_ROWS_SUB + half * (TBL_ROWS_SUB // 2)

        def build(r, c, _row0=row0, _half=half):
            row = _row0 + r
            i = lax.shift_right_logical(row, 6)
            j = lax.bitwise_and(row, 63)
            for c4 in range(4):
                bufs[_half][r, pl.ds(c4 * L, L)] = tab_v[i, pl.ds(c4 * L, L)]
                bufs[_half][r, pl.ds(EMB_DIM + c4 * L, L)] = (
                    tab_v[j, pl.ds(c4 * L, L)])
            return c

        lax.fori_loop(0, TBL_ROWS_SUB // 2, build, 0)
        pltpu.async_copy(
            bufs[half].at[pl.ds(0, TBL_ROWS_SUB // 2)],
            shared.at[pl.ds(row0, TBL_ROWS_SUB // 2)], stage_sem)

    lanes = lax.iota(jnp.int32, L)
    wvec = jnp.where(lax.bitwise_and(lanes, 1) == 0, NUM_BINS, 1)
    swap = lax.bitwise_xor(lanes, 1)
    compress = lax.shift_left(lax.bitwise_and(lanes, 7), 1)
    tailsel = lax.bitwise_and(lanes + 4, 7)

    def _pairs8(vec):
        c = vec * wvec
        s = c + c.at[swap].get(mode="promise_in_bounds")
        return s.at[compress].get(mode="promise_in_bounds")

    for h2 in range(2):
        pltpu.sync_copy(idx_hbm.at[wid, pl.ds(h2 * (ROWS_W // 2),
                                              ROWS_W // 2)], raw_v)

        def pair(r, c, _r0=h2 * (ROWS_W // 2)):
            for u in range(6):
                ga = _pairs8(raw_v[r, pl.ds(u * 2 * L, L)])
                gb = _pairs8(raw_v[r, pl.ds(u * 2 * L + L, L)])
                pidx_v[_r0 + r, pl.ds(u * L, L)] = jnp.where(
                    lanes < 8, ga, gb)
            gt = _pairs8(raw_v[r, pl.ds(SEQ - L, L)])
            pidx_v[_r0 + r, pl.ds(6 * L, L)] = (
                gt.at[tailsel].get(mode="promise_in_bounds"))
            return c

        lax.fori_loop(0, ROWS_W // 2, pair, 0)

    for half in range(2):
        pltpu.make_async_copy(
            bufs[half].at[pl.ds(0, TBL_ROWS_SUB // 2)],
            shared.at[pl.ds(sid * TBL_ROWS_SUB, TBL_ROWS_SUB // 2)],
            stage_sem).wait()
    plsc.subcore_barrier()

    def fire_gathers(s, b):
        for half in range(2):
            pltpu.async_copy(
                shared.at[pidx_v.at[2 * s + half, pl.ds(0, PAIR_R)]],
                bufs[b].at[pl.ds(half * PAIR_R, PAIR_R)], gsem)

    def drain_gathers(s, b):
        for half in range(2):
            pltpu.make_async_copy(
                shared.at[pidx_v.at[2 * s + half, pl.ds(0, PAIR_R)]],
                bufs[b].at[pl.ds(half * PAIR_R, PAIR_R)], gsem).wait()

    for b in range(2):
        fire_gathers(b, b)

    def step(jj, carry):
        for b in range(2):
            s = jj * 2 + b
            drain_gathers(s, b)
            pltpu.async_copy(
                bufs[b], out_hbm.at[pl.ds(base + s * 2 * PAIR_R,
                                          2 * PAIR_R)], ssem)

            @pl.when(jj < NSTEP // 2 - 1)
            def _():
                pltpu.make_async_copy(
                    bufs[b], out_hbm.at[pl.ds(base, 2 * PAIR_R)],
                    ssem).wait()
                fire_gathers(s + 2, b)
        return carry

    lax.fori_loop(0, NSTEP // 2, step, 0)

    for b in range(2):
        pltpu.make_async_copy(
            bufs[b], out_hbm.at[pl.ds(base, 2 * PAIR_R)], ssem).wait()


@functools.partial(jax.jit, static_argnames=())
def kernel(duration_bins, table):
    idx = duration_bins.astype(jnp.int32).reshape(NW, ROWS_W, SEQ)
    mesh = plsc.VectorSubcoreMesh(core_axis_name="c", subcore_axis_name="s")
    out = pl.kernel(
        _body,
        out_type=jax.ShapeDtypeStruct((P, 2 * EMB_DIM), jnp.float32),
        mesh=mesh,
        scratch_types=[
            pltpu.VMEM_SHARED((NUM_BINS * NUM_BINS, 2 * EMB_DIM),
                              jnp.float32),
            pltpu.VMEM((ROWS_W // 2, SEQ), jnp.int32),
            pltpu.VMEM((ROWS_W, PIDX_STRIDE), jnp.int32),
            pltpu.VMEM((NUM_BINS, EMB_DIM), jnp.float32),
            [pltpu.VMEM((2 * PAIR_R, 2 * EMB_DIM), jnp.float32)
             for _ in range(2)],
            pltpu.SemaphoreType.DMA,
            pltpu.SemaphoreType.DMA,
            pltpu.SemaphoreType.DMA,
        ],
    )(idx, table)
    return out.reshape(BATCH, SEQ, EMB_DIM)


# speedup vs baseline: 8.2389x; 1.6629x over previous
import functools

import jax
import jax.numpy as jnp
from jax import lax
from jax.experimental import pallas as pl
from jax.experimental.pallas import tpu as pltpu
from jax.experimental.pallas import tpu_sc as plsc

NUM_BINS = 64
EMB_DIM = 64
BATCH = 4096
SEQ = 200

NC = 2
NS = 16
NW = NC * NS
L = 16

B = BATCH * SEQ
B_PER_W = B // NW
CHUNK = 128
NSTEP = B_PER_W // CHUNK
NBUF = 4
ROW_W128 = 2 * EMB_DIM


def _body(idx_hbm, table_hbm, out_hbm, shared, idx_v, tab_v, bufs,
          stage_sem, gsem, ssem):
    sid = lax.axis_index("s")
    wid = sid * NC + lax.axis_index("c")
    base = wid * B_PER_W

    @pl.when(sid == 0)
    def _():
        pltpu.sync_copy(table_hbm, tab_v)

        def build(r, c):
            src = lax.bitwise_xor(r, 1)
            for c4 in range(EMB_DIM // L):
                row = tab_v[src, pl.ds(c4 * L, L)]
                bufs[0][r, pl.ds(c4 * L, L)] = row
                bufs[0][r, pl.ds(EMB_DIM + c4 * L, L)] = row
            return c

        lax.fori_loop(0, NUM_BINS, build, 0)
        pltpu.async_copy(bufs[0].at[pl.ds(0, NUM_BINS)], shared,
                         stage_sem).wait()

    pltpu.sync_copy(idx_hbm.at[wid], idx_v)
    plsc.subcore_barrier()

    for k in range(NBUF):
        pltpu.async_copy(shared.at[idx_v.at[k]], bufs[k], gsem)

    def step(jj, carry):
        for k in range(NBUF):
            j = jj * NBUF + k
            pltpu.make_async_copy(shared.at[idx_v.at[j]], bufs[k],
                                  gsem).wait()
            pltpu.async_copy(
                bufs[k], out_hbm.at[pl.ds(base + j * CHUNK, CHUNK)], ssem)

            @pl.when(jj < NSTEP // NBUF - 1)
            def _():
                pltpu.make_async_copy(
                    bufs[k], out_hbm.at[pl.ds(base, CHUNK)], ssem).wait()
                pltpu.async_copy(shared.at[idx_v.at[j + NBUF]], bufs[k],
                                 gsem)
        return carry

    lax.fori_loop(0, NSTEP // NBUF, step, 0)

    for k in range(NBUF):
        pltpu.make_async_copy(
            bufs[k], out_hbm.at[pl.ds(base, CHUNK)], ssem).wait()


@functools.partial(jax.jit, static_argnames=())
def kernel(duration_bins, table):
    idx = jnp.bitwise_xor(duration_bins.astype(jnp.int32), 1).reshape(
        NW, NSTEP, CHUNK)
    mesh = plsc.VectorSubcoreMesh(core_axis_name="c", subcore_axis_name="s")
    out = pl.kernel(
        _body,
        out_type=jax.ShapeDtypeStruct((B, ROW_W128), jnp.float32),
        mesh=mesh,
        scratch_types=[
            pltpu.VMEM_SHARED((NUM_BINS, ROW_W128), jnp.float32),
            pltpu.VMEM((NSTEP, CHUNK), jnp.int32),
            pltpu.VMEM((NUM_BINS, EMB_DIM), jnp.float32),
            [pltpu.VMEM((CHUNK, ROW_W128), jnp.float32)
             for _ in range(NBUF)],
            pltpu.SemaphoreType.DMA,
            pltpu.SemaphoreType.DMA,
            pltpu.SemaphoreType.DMA,
        ],
    )(idx, table)
    return out.reshape(BATCH, SEQ, ROW_W128)[:, :, :EMB_DIM]
